# baseline (speedup 1.0000x reference)
import functools

import jax
import jax.numpy as jnp
from jax import lax
from jax.experimental import pallas as pl
from jax.experimental.pallas import tpu as pltpu
from jax.experimental.pallas import tpu_sc as plsc

_B = 16
_H = 16
_S = 2048
_D = 64
_LANES = 16


def _scatter_rows(cache_ref, val_ref, pos_vec, buf, sem, h):
    b = lax.iota(jnp.int32, _LANES)
    src_rows = b * _H + h
    pltpu.async_copy(val_ref.at[src_rows], buf, sem).wait()
    dst_rows = src_rows * _S + pos_vec - 1
    pltpu.async_copy(buf, cache_ref.at[dst_rows], sem).wait()


def _sc_update(kc_ref, vc_ref, k_val, v_val, input_pos, pos_vmem, buf, sem):
    c = lax.axis_index("c")
    s = lax.axis_index("s")
    pltpu.sync_copy(input_pos, pos_vmem)
    pos_vec = pos_vmem[...]

    @pl.when(c == 0)
    def _():
        _scatter_rows(kc_ref, k_val, pos_vec, buf, sem, s)
        _scatter_rows(vc_ref, v_val, pos_vec, buf, sem, s)


_update = functools.partial(
    pl.kernel,
    mesh=plsc.VectorSubcoreMesh(
        core_axis_name="c", subcore_axis_name="s", num_cores=2, num_subcores=16
    ),
    scratch_types=[
        pltpu.VMEM((_LANES,), jnp.int32),
        pltpu.VMEM((_H, _D), jnp.float32),
        pltpu.SemaphoreType.DMA,
    ],
    compiler_params=pltpu.CompilerParams(use_tc_tiling_on_sc=False),
)(_sc_update)


def kernel(k_cache, v_cache, k_val, v_val, input_pos):
    flat = (_B * _H * _S, _D)
    kc_ref = jax.new_ref(k_cache.reshape(flat))
    vc_ref = jax.new_ref(v_cache.reshape(flat))
    _update(
        kc_ref,
        vc_ref,
        k_val.reshape(_B * _H, _D),
        v_val.reshape(_B * _H, _D),
        input_pos,
    )
    shape = (_B, _H, _S, _D)
    return kc_ref[...].reshape(shape), vc_ref[...].reshape(shape)


# speedup vs baseline: 1.0414x; 1.0414x over previous
import jax
import jax.numpy as jnp
from jax.experimental import pallas as pl
from jax.experimental.pallas import tpu as pltpu

_B = 16
_H = 16
_S = 2048
_D = 64


def _body(pos_ref, kc_ref, vc_ref, kval_ref, vval_ref, kout_ref, vout_ref):
    b = pl.program_id(0)
    r = pos_ref[b] - 1
    kout_ref[...] = kc_ref[...]
    vout_ref[...] = vc_ref[...]
    kout_ref[pl.ds(r, 1), :] = kval_ref[...]
    vout_ref[pl.ds(r, 1), :] = vval_ref[...]


def kernel(k_cache, v_cache, k_val, v_val, input_pos):
    cache_spec = pl.BlockSpec((None, None, _S, _D), lambda b, h, pos: (b, h, 0, 0))
    val_spec = pl.BlockSpec((None, None, 1, _D), lambda b, h, pos: (b, h, 0, 0))
    grid_spec = pltpu.PrefetchScalarGridSpec(
        num_scalar_prefetch=1,
        grid=(_B, _H),
        in_specs=[cache_spec, cache_spec, val_spec, val_spec],
        out_specs=[cache_spec, cache_spec],
    )
    out_shape = jax.ShapeDtypeStruct((_B, _H, _S, _D), jnp.float32)
    return pl.pallas_call(
        _body,
        grid_spec=grid_spec,
        out_shape=[out_shape, out_shape],
        compiler_params=pltpu.CompilerParams(
            dimension_semantics=("arbitrary", "arbitrary"),
        ),
    )(input_pos, k_cache, v_cache, k_val, v_val)
